# baseline (speedup 1.0000x reference)
import functools

import jax
import jax.numpy as jnp
from jax import lax
from jax.experimental import pallas as pl
from jax.experimental.pallas import tpu as pltpu
from jax.experimental.pallas import tpu_sc as plsc

_N_CLASS = 20
_S = 96
_CELLS = _S * _S
_ROWS = _CELLS // 128
_MAX_OUT = 30
_IOU_T = 0.4
_SCORE_T = 0.1
_NEG_INF = float("-inf")
_BIG = 1 << 30

_BD1 = _N_CLASS * _CELLS
_BD2 = _BD1 + 2 * _CELLS
_HALF = _CELLS // 2
_GROUPS = _HALF // 16


def _sc_body(net_hbm, out_hbm, inbuf, outbuf, sems):
    cid = lax.axis_index("c")
    sid = lax.axis_index("s")
    wid = sid * 2 + cid
    img = wid // 2
    half = wid % 2
    lanes = lax.broadcasted_iota(jnp.int32, (16,), 0)
    pending = [None, None]

    regions = ((0, _N_CLASS, 0), (_BD1, 2, _N_CLASS), (_BD2, 8, _N_CLASS + 2))
    for base, nch, chbase in regions:
        nwords = _HALF * nch
        src0 = base + half * nwords
        pltpu.sync_copy(net_hbm.at[img, pl.ds(src0, nwords)],
                        inbuf.at[pl.ds(0, nwords)])
        for c in range(nch):
            gc = chbase + c
            par = gc % 2
            step = jnp.int32(64 * nch)

            if pending[par] is not None:
                pending[par].wait()
                pending[par] = None

            def gbody(g, idx, _nch=nch, _par=par):
                for u in range(4):
                    v = plsc.load_gather(inbuf, [idx + (u * 16 * _nch)])
                    outbuf[_par, pl.ds(g * 64 + u * 16, 16)] = v
                return idx + step

            lax.fori_loop(0, _GROUPS // 4, gbody, lanes * nch + c)
            cp = pltpu.make_async_copy(
                outbuf.at[par],
                out_hbm.at[img, gc, pl.ds(half * _HALF, _HALF)],
                sems.at[par])
            cp.start()
            pending[par] = cp
    for par in range(2):
        if pending[par] is not None:
            pending[par].wait()


def _sc_deinterleave(net_outs):
    B = net_outs.shape[0]
    mesh = plsc.VectorSubcoreMesh(core_axis_name="c", subcore_axis_name="s")
    k = functools.partial(
        pl.kernel, mesh=mesh,
        compiler_params=pltpu.CompilerParams(needs_layout_passes=False),
        out_type=jax.ShapeDtypeStruct((B, 30, _CELLS), jnp.float32),
        scratch_types=[
            pltpu.VMEM((_HALF * _N_CLASS,), jnp.float32),
            pltpu.VMEM((2, _HALF), jnp.float32),
            pltpu.SemaphoreType.DMA((2,)),
        ],
    )(_sc_body)
    return k(net_outs)


def _rmax(a):
    return jnp.max(jnp.max(a, axis=2, keepdims=True), axis=1, keepdims=True)


def _rmin_i(a):
    return jnp.min(jnp.min(a, axis=2, keepdims=True), axis=1, keepdims=True)


def _rsum(a):
    return jnp.sum(jnp.sum(a, axis=2, keepdims=True), axis=1, keepdims=True)


def _nms_kernel(feat_ref, wf_ref, hf_ref, out_ref, rows_scr):
    B = feat_ref.shape[0]
    wf = wf_ref[...][None]
    hf = hf_ref[...][None]

    def split_decode(s):
        tx = feat_ref[:, _N_CLASS + 2 + 4 * s + 0]
        ty = feat_ref[:, _N_CLASS + 2 + 4 * s + 1]
        tw = feat_ref[:, _N_CLASS + 2 + 4 * s + 2]
        th = feat_ref[:, _N_CLASS + 2 + 4 * s + 3]
        xc = (tx + wf) / jnp.float32(_S)
        yc = (ty + hf) / jnp.float32(_S)
        wb = tw * tw
        hb = th * th
        y1 = yc - hb / 2.0
        x1 = xc - wb / 2.0
        y2 = yc + hb / 2.0
        x2 = xc + wb / 2.0
        area = (y2 - y1) * (x2 - x1)
        return y1, x1, y2, x2, area

    y1_0, x1_0, y2_0, x2_0, ar0 = split_decode(0)
    y1_1, x1_1, y2_1, x2_1, ar1 = split_decode(1)

    conf0 = feat_ref[:, _N_CLASS]
    conf1 = feat_ref[:, _N_CLASS + 1]
    p = feat_ref[:, 0]
    best0 = conf0 * p
    best1 = conf1 * p
    cl0 = jnp.zeros_like(best0)
    cl1 = jnp.zeros_like(best1)
    for c in range(1, _N_CLASS):
        p = feat_ref[:, c]
        q0 = conf0 * p
        q1 = conf1 * p
        cl0 = jnp.where(q0 > best0, jnp.float32(c), cl0)
        best0 = jnp.maximum(best0, q0)
        cl1 = jnp.where(q1 > best1, jnp.float32(c), cl1)
        best1 = jnp.maximum(best1, q1)
    sc0 = jnp.where(best0 >= _SCORE_T, best0, jnp.float32(0.0))
    sc1 = jnp.where(best1 >= _SCORE_T, best1, jnp.float32(0.0))

    cell = (jax.lax.broadcasted_iota(jnp.int32, (_ROWS, 128), 0) * 128
            + jax.lax.broadcasted_iota(jnp.int32, (_ROWS, 128), 1))
    nf0 = (cell * 2)[None]
    nf1 = (cell * 2 + 1)[None]

    rows_scr[...] = jnp.zeros((B, 32, 128), jnp.float32)

    def body(i, carry):
        w0, w1 = carry
        best = jnp.maximum(_rmax(w0), _rmax(w1))
        n0 = _rmin_i(jnp.where(w0 == best, nf0, _BIG))
        n1 = _rmin_i(jnp.where(w1 == best, nf1, _BIG))
        nwin = jnp.minimum(n0, n1)
        valid = best > _NEG_INF
        sel0 = nf0 == nwin
        sel1 = nf1 == nwin

        def ext(a0, a1):
            return _rsum(jnp.where(sel0, a0, 0.0) + jnp.where(sel1, a1, 0.0))

        y1b = ext(y1_0, y1_1)
        x1b = ext(x1_0, x1_1)
        y2b = ext(y2_0, y2_1)
        x2b = ext(x2_0, x2_1)
        clb = ext(cl0, cl1)
        a1b = (y2b - y1b) * (x2b - x1b)

        def supp(w, ys1, xs1, ys2, xs2, a2, sels):
            yi1 = jnp.maximum(y1b, ys1)
            xi1 = jnp.maximum(x1b, xs1)
            yi2 = jnp.minimum(y2b, ys2)
            xi2 = jnp.minimum(x2b, xs2)
            inter = (jnp.maximum(yi2 - yi1, 0.0)
                     * jnp.maximum(xi2 - xi1, 0.0))
            denom = jnp.maximum(a1b + a2 - inter, jnp.float32(1e-9))
            iou = inter / denom
            kill = jnp.logical_and(valid,
                                   jnp.logical_or(iou > _IOU_T, sels))
            return jnp.where(kill, _NEG_INF, w)

        w0 = supp(w0, y1_0, x1_0, y2_0, x2_0, ar0, sel0)
        w1 = supp(w1, y1_1, x1_1, y2_1, x2_1, ar1, sel1)

        scb = jnp.where(valid, best, jnp.float32(0.0))
        row6 = jnp.concatenate([y1b, x1b, y2b, x2b, scb, clb], axis=2)
        row6 = jnp.where(valid, row6, jnp.float32(0.0))
        rowp = jnp.concatenate(
            [row6, jnp.zeros((B, 1, 122), jnp.float32)], axis=2)
        rows_scr[:, pl.ds(i, 1), :] = rowp
        return (w0, w1)

    jax.lax.fori_loop(0, _MAX_OUT, body, (sc0, sc1))

    rows_all = rows_scr[...]
    lane = jax.lax.broadcasted_iota(jnp.int32, (1, 1, 128), 2)
    scores = jnp.sum(jnp.where(lane == 4, rows_all, 0.0),
                     axis=2, keepdims=True)
    col = jax.lax.broadcasted_iota(jnp.int32, (1, 32, 1), 1)

    def sbody(k, sc):
        smax = jnp.max(sc, axis=1, keepdims=True)
        kidx = jnp.min(jnp.where(sc == smax, col, jnp.int32(99)),
                       axis=1, keepdims=True)
        selk = col == kidx
        outrow = jnp.sum(jnp.where(selk, rows_all, 0.0),
                         axis=1, keepdims=True)
        out_ref[:, pl.ds(k, 1), :] = outrow
        return jnp.where(selk, jnp.float32(-1.0), sc)

    jax.lax.fori_loop(0, _MAX_OUT, sbody, scores)


def _mesh_consts():
    cellv = jnp.arange(_CELLS, dtype=jnp.int32)
    wf = (cellv % _S).astype(jnp.float32).reshape(_ROWS, 128)
    hf = (cellv // _S).astype(jnp.float32).reshape(_ROWS, 128)
    return wf, hf


def kernel(net_outs):
    B = net_outs.shape[0]
    feat = _sc_deinterleave(net_outs).reshape(B, 30, _ROWS, 128)
    wf, hf = _mesh_consts()
    out = pl.pallas_call(
        _nms_kernel,
        out_shape=jax.ShapeDtypeStruct((B, 32, 128), jnp.float32),
        scratch_shapes=[pltpu.VMEM((B, 32, 128), jnp.float32)],
    )(feat, wf, hf)
    return out[:, :_MAX_OUT, :6]


# speedup vs baseline: 1.0072x; 1.0072x over previous
import functools

import jax
import jax.numpy as jnp
from jax import lax
from jax.experimental import pallas as pl
from jax.experimental.pallas import tpu as pltpu
from jax.experimental.pallas import tpu_sc as plsc

_N_CLASS = 20
_S = 96
_CELLS = _S * _S
_ROWS = _CELLS // 128
_MAX_OUT = 30
_IOU_T = 0.4
_SCORE_T = 0.1
_NEG_INF = float("-inf")
_BIG = 1 << 30

_BD1 = _N_CLASS * _CELLS
_BD2 = _BD1 + 2 * _CELLS
_HALF = _CELLS // 2
_GROUPS = _HALF // 16


def _sc_body(net_hbm, out_hbm, inbuf, outbuf, sems):
    cid = lax.axis_index("c")
    sid = lax.axis_index("s")
    wid = sid * 2 + cid
    img = wid // 2
    half = wid % 2
    lanes = lax.broadcasted_iota(jnp.int32, (16,), 0)

    regions = ((0, _N_CLASS, 0), (_BD1, 2, _N_CLASS), (_BD2, 8, _N_CLASS + 2))
    for base, nch, chbase in regions:
        nwords = _HALF * nch
        src0 = base + half * nwords
        pltpu.sync_copy(net_hbm.at[img, pl.ds(src0, nwords)],
                        inbuf.at[pl.ds(0, nwords)])
        for c in range(nch):
            step = jnp.int32(16 * nch)

            def gbody(g, idx, _nch=nch):
                v = plsc.load_gather(inbuf, [idx])
                outbuf[0, pl.ds(g * 16, 16)] = v
                return idx + step

            lax.fori_loop(0, _GROUPS, gbody, lanes * nch + c)
            pltpu.sync_copy(
                outbuf.at[0],
                out_hbm.at[img, chbase + c, pl.ds(half * _HALF, _HALF)])


def _sc_deinterleave(net_outs):
    B = net_outs.shape[0]
    mesh = plsc.VectorSubcoreMesh(core_axis_name="c", subcore_axis_name="s")
    k = functools.partial(
        pl.kernel, mesh=mesh,
        compiler_params=pltpu.CompilerParams(needs_layout_passes=False),
        out_type=jax.ShapeDtypeStruct((B, 30, _CELLS), jnp.float32),
        scratch_types=[
            pltpu.VMEM((_HALF * _N_CLASS,), jnp.float32),
            pltpu.VMEM((2, _HALF), jnp.float32),
            pltpu.SemaphoreType.DMA((2,)),
        ],
    )(_sc_body)
    return k(net_outs)


def _rmax(a):
    return jnp.max(jnp.max(a, axis=2, keepdims=True), axis=1, keepdims=True)


def _rmin_i(a):
    return jnp.min(jnp.min(a, axis=2, keepdims=True), axis=1, keepdims=True)


def _rsum(a):
    return jnp.sum(jnp.sum(a, axis=2, keepdims=True), axis=1, keepdims=True)


def _nms_kernel(feat_ref, wf_ref, hf_ref, out_ref, rows_scr):
    B = feat_ref.shape[0]
    wf = wf_ref[...][None]
    hf = hf_ref[...][None]

    def split_decode(s):
        tx = feat_ref[:, _N_CLASS + 2 + 4 * s + 0]
        ty = feat_ref[:, _N_CLASS + 2 + 4 * s + 1]
        tw = feat_ref[:, _N_CLASS + 2 + 4 * s + 2]
        th = feat_ref[:, _N_CLASS + 2 + 4 * s + 3]
        xc = (tx + wf) / jnp.float32(_S)
        yc = (ty + hf) / jnp.float32(_S)
        wb = tw * tw
        hb = th * th
        y1 = yc - hb / 2.0
        x1 = xc - wb / 2.0
        y2 = yc + hb / 2.0
        x2 = xc + wb / 2.0
        area = (y2 - y1) * (x2 - x1)
        return y1, x1, y2, x2, area

    y1_0, x1_0, y2_0, x2_0, ar0 = split_decode(0)
    y1_1, x1_1, y2_1, x2_1, ar1 = split_decode(1)

    conf0 = feat_ref[:, _N_CLASS]
    conf1 = feat_ref[:, _N_CLASS + 1]
    p = feat_ref[:, 0]
    best0 = conf0 * p
    best1 = conf1 * p
    cl0 = jnp.zeros_like(best0)
    cl1 = jnp.zeros_like(best1)
    for c in range(1, _N_CLASS):
        p = feat_ref[:, c]
        q0 = conf0 * p
        q1 = conf1 * p
        cl0 = jnp.where(q0 > best0, jnp.float32(c), cl0)
        best0 = jnp.maximum(best0, q0)
        cl1 = jnp.where(q1 > best1, jnp.float32(c), cl1)
        best1 = jnp.maximum(best1, q1)
    sc0 = jnp.where(best0 >= _SCORE_T, best0, jnp.float32(0.0))
    sc1 = jnp.where(best1 >= _SCORE_T, best1, jnp.float32(0.0))

    cell = (jax.lax.broadcasted_iota(jnp.int32, (_ROWS, 128), 0) * 128
            + jax.lax.broadcasted_iota(jnp.int32, (_ROWS, 128), 1))
    nf0 = (cell * 2)[None]
    nf1 = (cell * 2 + 1)[None]

    rows_scr[...] = jnp.zeros((B, 32, 128), jnp.float32)

    def body(i, carry):
        w0, w1 = carry
        best = jnp.maximum(_rmax(w0), _rmax(w1))
        n0 = _rmin_i(jnp.where(w0 == best, nf0, _BIG))
        n1 = _rmin_i(jnp.where(w1 == best, nf1, _BIG))
        nwin = jnp.minimum(n0, n1)
        valid = best > _NEG_INF
        sel0 = nf0 == nwin
        sel1 = nf1 == nwin

        def ext(a0, a1):
            return _rsum(jnp.where(sel0, a0, 0.0) + jnp.where(sel1, a1, 0.0))

        y1b = ext(y1_0, y1_1)
        x1b = ext(x1_0, x1_1)
        y2b = ext(y2_0, y2_1)
        x2b = ext(x2_0, x2_1)
        clb = ext(cl0, cl1)
        a1b = (y2b - y1b) * (x2b - x1b)

        def supp(w, ys1, xs1, ys2, xs2, a2, sels):
            yi1 = jnp.maximum(y1b, ys1)
            xi1 = jnp.maximum(x1b, xs1)
            yi2 = jnp.minimum(y2b, ys2)
            xi2 = jnp.minimum(x2b, xs2)
            inter = (jnp.maximum(yi2 - yi1, 0.0)
                     * jnp.maximum(xi2 - xi1, 0.0))
            denom = jnp.maximum(a1b + a2 - inter, jnp.float32(1e-9))
            iou = inter / denom
            kill = jnp.logical_and(valid,
                                   jnp.logical_or(iou > _IOU_T, sels))
            return jnp.where(kill, _NEG_INF, w)

        w0 = supp(w0, y1_0, x1_0, y2_0, x2_0, ar0, sel0)
        w1 = supp(w1, y1_1, x1_1, y2_1, x2_1, ar1, sel1)

        scb = jnp.where(valid, best, jnp.float32(0.0))
        row6 = jnp.concatenate([y1b, x1b, y2b, x2b, scb, clb], axis=2)
        row6 = jnp.where(valid, row6, jnp.float32(0.0))
        rowp = jnp.concatenate(
            [row6, jnp.zeros((B, 1, 122), jnp.float32)], axis=2)
        rows_scr[:, pl.ds(i, 1), :] = rowp
        return (w0, w1)

    jax.lax.fori_loop(0, _MAX_OUT, body, (sc0, sc1))

    rows_all = rows_scr[...]
    lane = jax.lax.broadcasted_iota(jnp.int32, (1, 1, 128), 2)
    scores = jnp.sum(jnp.where(lane == 4, rows_all, 0.0),
                     axis=2, keepdims=True)
    col = jax.lax.broadcasted_iota(jnp.int32, (1, 32, 1), 1)

    def sbody(k, sc):
        smax = jnp.max(sc, axis=1, keepdims=True)
        kidx = jnp.min(jnp.where(sc == smax, col, jnp.int32(99)),
                       axis=1, keepdims=True)
        selk = col == kidx
        outrow = jnp.sum(jnp.where(selk, rows_all, 0.0),
                         axis=1, keepdims=True)
        out_ref[:, pl.ds(k, 1), :] = outrow
        return jnp.where(selk, jnp.float32(-1.0), sc)

    jax.lax.fori_loop(0, _MAX_OUT, sbody, scores)


def _mesh_consts():
    cellv = jnp.arange(_CELLS, dtype=jnp.int32)
    wf = (cellv % _S).astype(jnp.float32).reshape(_ROWS, 128)
    hf = (cellv // _S).astype(jnp.float32).reshape(_ROWS, 128)
    return wf, hf


def kernel(net_outs):
    B = net_outs.shape[0]
    feat = _sc_deinterleave(net_outs).reshape(B, 30, _ROWS, 128)
    wf, hf = _mesh_consts()
    out = pl.pallas_call(
        _nms_kernel,
        out_shape=jax.ShapeDtypeStruct((B, 32, 128), jnp.float32),
        scratch_shapes=[pltpu.VMEM((B, 32, 128), jnp.float32)],
    )(feat, wf, hf)
    return out[:, :_MAX_OUT, :6]


# speedup vs baseline: 1.0442x; 1.0367x over previous
import functools

import jax
import jax.numpy as jnp
from jax import lax
from jax.experimental import pallas as pl
from jax.experimental.pallas import tpu as pltpu
from jax.experimental.pallas import tpu_sc as plsc

_N_CLASS = 20
_S = 96
_CELLS = _S * _S
_ROWS = _CELLS // 128
_MAX_OUT = 30
_IOU_T = 0.4
_SCORE_T = 0.1
_NEG_INF = float("-inf")
_BIG = 1 << 30

_BD1 = _N_CLASS * _CELLS
_BD2 = _BD1 + 2 * _CELLS
_HALF = _CELLS // 2
_GROUPS = _HALF // 16


def _sc_body(net_hbm, out_hbm, inbuf, outbuf, sem):
    cid = lax.axis_index("c")
    sid = lax.axis_index("s")
    wid = sid * 2 + cid
    img = wid // 2
    half = wid % 2
    lanes = lax.broadcasted_iota(jnp.int32, (16,), 0)

    regions = ((0, _N_CLASS, 0), (_BD1, 2, _N_CLASS), (_BD2, 8, _N_CLASS + 2))
    for base, nch, chbase in regions:
        nwords = _HALF * nch
        src0 = base + half * nwords
        pltpu.sync_copy(net_hbm.at[img, pl.ds(src0, nwords)],
                        inbuf.at[pl.ds(0, nwords)])
        for c in range(nch):
            step = jnp.int32(16 * nch)

            def gbody(g, idx, _c=c, _nch=nch):
                v = plsc.load_gather(inbuf, [idx])
                outbuf[pl.ds(g * 16, 16)] = v
                return idx + step

            lax.fori_loop(0, _GROUPS, gbody, lanes * nch + c)
            pltpu.sync_copy(
                outbuf,
                out_hbm.at[img, chbase + c, pl.ds(half * _HALF, _HALF)])


def _sc_deinterleave(net_outs):
    B = net_outs.shape[0]
    mesh = plsc.VectorSubcoreMesh(core_axis_name="c", subcore_axis_name="s")
    k = functools.partial(
        pl.kernel, mesh=mesh,
        compiler_params=pltpu.CompilerParams(needs_layout_passes=False),
        out_type=jax.ShapeDtypeStruct((B, 30, _CELLS), jnp.float32),
        scratch_types=[
            pltpu.VMEM((_HALF * _N_CLASS,), jnp.float32),
            pltpu.VMEM((_HALF,), jnp.float32),
            pltpu.SemaphoreType.DMA,
        ],
    )(_sc_body)
    return k(net_outs)


def _rmax(a):
    return jnp.max(jnp.max(a, axis=2, keepdims=True), axis=1, keepdims=True)


def _rmin_i(a):
    return jnp.min(jnp.min(a, axis=2, keepdims=True), axis=1, keepdims=True)


def _rsum(a):
    return jnp.sum(jnp.sum(a, axis=2, keepdims=True), axis=1, keepdims=True)


def _nms_kernel(feat_ref, wf_ref, hf_ref, out_ref, rows_scr):
    B = feat_ref.shape[0]
    wf = wf_ref[...][None]
    hf = hf_ref[...][None]

    def split_decode(s):
        conf = feat_ref[:, _N_CLASS + s]
        tx = feat_ref[:, _N_CLASS + 2 + 4 * s + 0]
        ty = feat_ref[:, _N_CLASS + 2 + 4 * s + 1]
        tw = feat_ref[:, _N_CLASS + 2 + 4 * s + 2]
        th = feat_ref[:, _N_CLASS + 2 + 4 * s + 3]
        xc = (tx + wf) / jnp.float32(_S)
        yc = (ty + hf) / jnp.float32(_S)
        wb = tw * tw
        hb = th * th
        y1 = yc - hb / 2.0
        x1 = xc - wb / 2.0
        y2 = yc + hb / 2.0
        x2 = xc + wb / 2.0
        best = conf * feat_ref[:, 0]
        cls = jnp.zeros_like(best)
        for c in range(1, _N_CLASS):
            p = conf * feat_ref[:, c]
            cls = jnp.where(p > best, jnp.float32(c), cls)
            best = jnp.maximum(best, p)
        score = jnp.where(best >= _SCORE_T, best, jnp.float32(0.0))
        area = (y2 - y1) * (x2 - x1)
        return y1, x1, y2, x2, score, cls, area

    y1_0, x1_0, y2_0, x2_0, sc0, cl0, ar0 = split_decode(0)
    y1_1, x1_1, y2_1, x2_1, sc1, cl1, ar1 = split_decode(1)

    rowi = jax.lax.broadcasted_iota(jnp.int32, (1, _ROWS, 1), 1)
    lanei = jax.lax.broadcasted_iota(jnp.int32, (1, 1, 128), 2)

    rows_scr[...] = jnp.zeros((B, 32, 128), jnp.float32)

    def argmax2(w):
        rowmax = jnp.max(w, axis=2, keepdims=True)
        bst = jnp.max(rowmax, axis=1, keepdims=True)
        r = jnp.min(jnp.where(rowmax == bst, rowi, _BIG),
                    axis=1, keepdims=True)
        rowsel = rowi == r
        lanevals = jnp.sum(jnp.where(rowsel, w, 0.0),
                           axis=1, keepdims=True)
        lane = jnp.min(jnp.where(lanevals == bst, lanei, _BIG),
                       axis=2, keepdims=True)
        return bst, r, lane, rowsel

    def body(i, carry):
        w0, w1 = carry
        best0, r0, l0, rowsel0 = argmax2(w0)
        best1, r1, l1, rowsel1 = argmax2(w1)
        best = jnp.maximum(best0, best1)
        n0 = jnp.where(best0 == best, (r0 * 128 + l0) * 2, _BIG)
        n1 = jnp.where(best1 == best, (r1 * 128 + l1) * 2 + 1, _BIG)
        nwin = jnp.minimum(n0, n1)
        pick0 = nwin == n0
        valid = best > _NEG_INF
        lanesel0 = lanei == l0
        lanesel1 = lanei == l1

        def ext(a0, a1):
            rv0 = jnp.sum(jnp.where(rowsel0, a0, 0.0), axis=1, keepdims=True)
            rv1 = jnp.sum(jnp.where(rowsel1, a1, 0.0), axis=1, keepdims=True)
            v0 = jnp.sum(jnp.where(lanesel0, rv0, 0.0), axis=2, keepdims=True)
            v1 = jnp.sum(jnp.where(lanesel1, rv1, 0.0), axis=2, keepdims=True)
            return jnp.where(pick0, v0, v1)

        y1b = ext(y1_0, y1_1)
        x1b = ext(x1_0, x1_1)
        y2b = ext(y2_0, y2_1)
        x2b = ext(x2_0, x2_1)
        clb = ext(cl0, cl1)
        a1b = (y2b - y1b) * (x2b - x1b)

        def supp(w, ys1, xs1, ys2, xs2, a2, picks, rowsel, lanesel):
            yi1 = jnp.maximum(y1b, ys1)
            xi1 = jnp.maximum(x1b, xs1)
            yi2 = jnp.minimum(y2b, ys2)
            xi2 = jnp.minimum(x2b, xs2)
            inter = (jnp.maximum(yi2 - yi1, 0.0)
                     * jnp.maximum(xi2 - xi1, 0.0))
            denom = jnp.maximum(a1b + a2 - inter, jnp.float32(1e-9))
            iou = inter / denom
            selfm = jnp.logical_and(picks,
                                    jnp.logical_and(rowsel, lanesel))
            kill = jnp.logical_and(valid,
                                   jnp.logical_or(iou > _IOU_T, selfm))
            return jnp.where(kill, _NEG_INF, w)

        w0 = supp(w0, y1_0, x1_0, y2_0, x2_0, ar0,
                  pick0, rowsel0, lanesel0)
        w1 = supp(w1, y1_1, x1_1, y2_1, x2_1, ar1,
                  jnp.logical_not(pick0), rowsel1, lanesel1)

        scb = jnp.where(valid, best, jnp.float32(0.0))
        row6 = jnp.concatenate([y1b, x1b, y2b, x2b, scb, clb], axis=2)
        row6 = jnp.where(valid, row6, jnp.float32(0.0))
        rowp = jnp.concatenate(
            [row6, jnp.zeros((B, 1, 122), jnp.float32)], axis=2)
        rows_scr[:, pl.ds(i, 1), :] = rowp
        return (w0, w1)

    jax.lax.fori_loop(0, _MAX_OUT, body, (sc0, sc1))

    rows_all = rows_scr[...]
    lane = jax.lax.broadcasted_iota(jnp.int32, (1, 1, 128), 2)
    scores = jnp.sum(jnp.where(lane == 4, rows_all, 0.0),
                     axis=2, keepdims=True)
    col = jax.lax.broadcasted_iota(jnp.int32, (1, 32, 1), 1)

    def sbody(k, sc):
        smax = jnp.max(sc, axis=1, keepdims=True)
        kidx = jnp.min(jnp.where(sc == smax, col, jnp.int32(99)),
                       axis=1, keepdims=True)
        selk = col == kidx
        outrow = jnp.sum(jnp.where(selk, rows_all, 0.0),
                         axis=1, keepdims=True)
        out_ref[:, pl.ds(k, 1), :] = outrow
        return jnp.where(selk, jnp.float32(-1.0), sc)

    jax.lax.fori_loop(0, _MAX_OUT, sbody, scores)


def _mesh_consts():
    cellv = jnp.arange(_CELLS, dtype=jnp.int32)
    wf = (cellv % _S).astype(jnp.float32).reshape(_ROWS, 128)
    hf = (cellv // _S).astype(jnp.float32).reshape(_ROWS, 128)
    return wf, hf


def kernel(net_outs):
    B = net_outs.shape[0]
    feat = _sc_deinterleave(net_outs).reshape(B, 30, _ROWS, 128)
    wf, hf = _mesh_consts()
    out = pl.pallas_call(
        _nms_kernel,
        out_shape=jax.ShapeDtypeStruct((B, 32, 128), jnp.float32),
        scratch_shapes=[pltpu.VMEM((B, 32, 128), jnp.float32)],
    )(feat, wf, hf)
    return out[:, :_MAX_OUT, :6]


# speedup vs baseline: 1.0631x; 1.0181x over previous
import functools

import jax
import jax.numpy as jnp
from jax import lax
from jax.experimental import pallas as pl
from jax.experimental.pallas import tpu as pltpu
from jax.experimental.pallas import tpu_sc as plsc

_N_CLASS = 20
_S = 96
_CELLS = _S * _S
_ROWS = _CELLS // 128
_MAX_OUT = 30
_IOU_T = 0.4
_SCORE_T = 0.1
_NEG_INF = float("-inf")
_BIG = 1 << 30

_BD1 = _N_CLASS * _CELLS
_BD2 = _BD1 + 2 * _CELLS
_HALF = _CELLS // 2
_GROUPS = _HALF // 16


def _sc_body(net_hbm, out_hbm, inbuf, outbuf, insem, outsems):
    cid = lax.axis_index("c")
    sid = lax.axis_index("s")
    wid = sid * 2 + cid
    img = wid // 2
    half = wid % 2
    lanes = lax.broadcasted_iota(jnp.int32, (16,), 0)

    regions = ((0, _N_CLASS, 0, 0),
               (_BD1, 2, _N_CLASS, _HALF * _N_CLASS),
               (_BD2, 8, _N_CLASS + 2, 0))

    def in_copy(ridx):
        base, nch, _, ioff = regions[ridx]
        nwords = _HALF * nch
        cp = pltpu.make_async_copy(
            net_hbm.at[img, pl.ds(base + half * nwords, nwords)],
            inbuf.at[pl.ds(ioff, nwords)], insem)
        cp.start()
        return cp

    incps = [in_copy(0), in_copy(1), None]
    pending = [None, None]

    for ridx, (base, nch, chbase, ioff) in enumerate(regions):
        if ridx == 1:
            incps[2] = in_copy(2)
        incps[ridx].wait()
        for c in range(nch):
            gc = chbase + c
            par = gc % 2
            step = jnp.int32(16 * nch)
            if pending[par] is not None:
                pending[par].wait()
                pending[par] = None

            def gbody(g, idx, _par=par):
                v = plsc.load_gather(inbuf, [idx])
                outbuf[_par, pl.ds(g * 16, 16)] = v
                return idx + step

            lax.fori_loop(0, _GROUPS, gbody, lanes * nch + c + ioff)
            cp = pltpu.make_async_copy(
                outbuf.at[par],
                out_hbm.at[img, gc, pl.ds(half * _HALF, _HALF)],
                outsems.at[par])
            cp.start()
            pending[par] = cp
    for par in range(2):
        if pending[par] is not None:
            pending[par].wait()


def _sc_deinterleave(net_outs):
    B = net_outs.shape[0]
    mesh = plsc.VectorSubcoreMesh(core_axis_name="c", subcore_axis_name="s")
    k = functools.partial(
        pl.kernel, mesh=mesh,
        compiler_params=pltpu.CompilerParams(needs_layout_passes=False),
        out_type=jax.ShapeDtypeStruct((B, 30, _CELLS), jnp.float32),
        scratch_types=[
            pltpu.VMEM((_HALF * (_N_CLASS + 2),), jnp.float32),
            pltpu.VMEM((2, _HALF), jnp.float32),
            pltpu.SemaphoreType.DMA,
            pltpu.SemaphoreType.DMA((2,)),
        ],
    )(_sc_body)
    return k(net_outs)


def _rmax(a):
    return jnp.max(jnp.max(a, axis=2, keepdims=True), axis=1, keepdims=True)


def _rmin_i(a):
    return jnp.min(jnp.min(a, axis=2, keepdims=True), axis=1, keepdims=True)


def _rsum(a):
    return jnp.sum(jnp.sum(a, axis=2, keepdims=True), axis=1, keepdims=True)


def _nms_kernel(feat_ref, wf_ref, hf_ref, out_ref, rows_scr):
    B = feat_ref.shape[0]
    wf = wf_ref[...][None]
    hf = hf_ref[...][None]

    def split_decode(s):
        conf = feat_ref[:, _N_CLASS + s]
        tx = feat_ref[:, _N_CLASS + 2 + 4 * s + 0]
        ty = feat_ref[:, _N_CLASS + 2 + 4 * s + 1]
        tw = feat_ref[:, _N_CLASS + 2 + 4 * s + 2]
        th = feat_ref[:, _N_CLASS + 2 + 4 * s + 3]
        xc = (tx + wf) / jnp.float32(_S)
        yc = (ty + hf) / jnp.float32(_S)
        wb = tw * tw
        hb = th * th
        y1 = yc - hb / 2.0
        x1 = xc - wb / 2.0
        y2 = yc + hb / 2.0
        x2 = xc + wb / 2.0
        best = conf * feat_ref[:, 0]
        cls = jnp.zeros_like(best)
        for c in range(1, _N_CLASS):
            p = conf * feat_ref[:, c]
            cls = jnp.where(p > best, jnp.float32(c), cls)
            best = jnp.maximum(best, p)
        score = jnp.where(best >= _SCORE_T, best, jnp.float32(0.0))
        area = (y2 - y1) * (x2 - x1)
        return y1, x1, y2, x2, score, cls, area

    y1_0, x1_0, y2_0, x2_0, sc0, cl0, ar0 = split_decode(0)
    y1_1, x1_1, y2_1, x2_1, sc1, cl1, ar1 = split_decode(1)

    rowi = jax.lax.broadcasted_iota(jnp.int32, (1, _ROWS, 1), 1)
    lanei = jax.lax.broadcasted_iota(jnp.int32, (1, 1, 128), 2)

    rows_scr[...] = jnp.zeros((B, 32, 128), jnp.float32)

    def argmax2(w):
        rowmax = jnp.max(w, axis=2, keepdims=True)
        bst = jnp.max(rowmax, axis=1, keepdims=True)
        r = jnp.min(jnp.where(rowmax == bst, rowi, _BIG),
                    axis=1, keepdims=True)
        rowsel = rowi == r
        lanevals = jnp.sum(jnp.where(rowsel, w, 0.0),
                           axis=1, keepdims=True)
        lane = jnp.min(jnp.where(lanevals == bst, lanei, _BIG),
                       axis=2, keepdims=True)
        return bst, r, lane, rowsel

    def body(i, carry):
        w0, w1 = carry
        best0, r0, l0, rowsel0 = argmax2(w0)
        best1, r1, l1, rowsel1 = argmax2(w1)
        best = jnp.maximum(best0, best1)
        n0 = jnp.where(best0 == best, (r0 * 128 + l0) * 2, _BIG)
        n1 = jnp.where(best1 == best, (r1 * 128 + l1) * 2 + 1, _BIG)
        nwin = jnp.minimum(n0, n1)
        pick0 = nwin == n0
        valid = best > _NEG_INF
        lanesel0 = lanei == l0
        lanesel1 = lanei == l1

        def ext(a0, a1):
            rv0 = jnp.sum(jnp.where(rowsel0, a0, 0.0), axis=1, keepdims=True)
            rv1 = jnp.sum(jnp.where(rowsel1, a1, 0.0), axis=1, keepdims=True)
            v0 = jnp.sum(jnp.where(lanesel0, rv0, 0.0), axis=2, keepdims=True)
            v1 = jnp.sum(jnp.where(lanesel1, rv1, 0.0), axis=2, keepdims=True)
            return jnp.where(pick0, v0, v1)

        y1b = ext(y1_0, y1_1)
        x1b = ext(x1_0, x1_1)
        y2b = ext(y2_0, y2_1)
        x2b = ext(x2_0, x2_1)
        clb = ext(cl0, cl1)
        a1b = (y2b - y1b) * (x2b - x1b)

        def supp(w, ys1, xs1, ys2, xs2, a2, picks, rowsel, lanesel):
            yi1 = jnp.maximum(y1b, ys1)
            xi1 = jnp.maximum(x1b, xs1)
            yi2 = jnp.minimum(y2b, ys2)
            xi2 = jnp.minimum(x2b, xs2)
            inter = (jnp.maximum(yi2 - yi1, 0.0)
                     * jnp.maximum(xi2 - xi1, 0.0))
            denom = jnp.maximum(a1b + a2 - inter, jnp.float32(1e-9))
            iou = inter / denom
            selfm = jnp.logical_and(picks,
                                    jnp.logical_and(rowsel, lanesel))
            kill = jnp.logical_and(valid,
                                   jnp.logical_or(iou > _IOU_T, selfm))
            return jnp.where(kill, _NEG_INF, w)

        w0 = supp(w0, y1_0, x1_0, y2_0, x2_0, ar0,
                  pick0, rowsel0, lanesel0)
        w1 = supp(w1, y1_1, x1_1, y2_1, x2_1, ar1,
                  jnp.logical_not(pick0), rowsel1, lanesel1)

        scb = jnp.where(valid, best, jnp.float32(0.0))
        row6 = jnp.concatenate([y1b, x1b, y2b, x2b, scb, clb], axis=2)
        row6 = jnp.where(valid, row6, jnp.float32(0.0))
        rowp = jnp.concatenate(
            [row6, jnp.zeros((B, 1, 122), jnp.float32)], axis=2)
        rows_scr[:, pl.ds(i, 1), :] = rowp
        return (w0, w1)

    jax.lax.fori_loop(0, _MAX_OUT, body, (sc0, sc1))

    rows_all = rows_scr[...]
    lane = jax.lax.broadcasted_iota(jnp.int32, (1, 1, 128), 2)
    scores = jnp.sum(jnp.where(lane == 4, rows_all, 0.0),
                     axis=2, keepdims=True)
    col = jax.lax.broadcasted_iota(jnp.int32, (1, 32, 1), 1)

    def sbody(k, sc):
        smax = jnp.max(sc, axis=1, keepdims=True)
        kidx = jnp.min(jnp.where(sc == smax, col, jnp.int32(99)),
                       axis=1, keepdims=True)
        selk = col == kidx
        outrow = jnp.sum(jnp.where(selk, rows_all, 0.0),
                         axis=1, keepdims=True)
        out_ref[:, pl.ds(k, 1), :] = outrow
        return jnp.where(selk, jnp.float32(-1.0), sc)

    jax.lax.fori_loop(0, _MAX_OUT, sbody, scores)


def _mesh_consts():
    cellv = jnp.arange(_CELLS, dtype=jnp.int32)
    wf = (cellv % _S).astype(jnp.float32).reshape(_ROWS, 128)
    hf = (cellv // _S).astype(jnp.float32).reshape(_ROWS, 128)
    return wf, hf


def kernel(net_outs):
    B = net_outs.shape[0]
    feat = _sc_deinterleave(net_outs).reshape(B, 30, _ROWS, 128)
    wf, hf = _mesh_consts()
    out = pl.pallas_call(
        _nms_kernel,
        out_shape=jax.ShapeDtypeStruct((B, 32, 128), jnp.float32),
        scratch_shapes=[pltpu.VMEM((B, 32, 128), jnp.float32)],
    )(feat, wf, hf)
    return out[:, :_MAX_OUT, :6]
